# baseline (speedup 1.0000x reference)
import jax
import jax.numpy as jnp
from jax import lax
from jax.experimental import pallas as pl
from jax.experimental.pallas import tpu as pltpu
from jax.experimental.pallas import tpu_sc as plsc

N = 10000
E = 320000
D = 128
COUT = 64
DP = 144
CH = 32
NCORES = 2
NSUB = 16
NTILES = NCORES * NSUB
NG = 324
NQ = NG // 12
EPAD = NTILES * NG * CH
NCHP = EPAD // CH
NR = 6
NI = 12
NPAD = 10112
RPT = NPAD // NSUB
BNP = 1264
BN = 1000


def _prep_body(x_ref, c12_ref, xpad_ref, atab_ref):
    xb = x_ref[...]
    ones = jnp.ones((BNP, 1), jnp.float32)
    zeros = jnp.zeros((BNP, DP - D - 1), jnp.float32)
    xpad_ref[...] = jnp.concatenate([xb, ones, zeros], axis=1)
    atab_ref[...] = jnp.dot(xb, c12_ref[...], preferred_element_type=jnp.float32)


def _finish_body(part_ref, wfc_ref, b_ref, out_ref):
    s = part_ref[0] + part_ref[1]
    h = s[:, :D]
    den = s[:, D:D + 1]
    h = h * (1.0 / (den + 1e-9))
    out_ref[...] = (
        jnp.dot(h, wfc_ref[...], preferred_element_type=jnp.float32) + b_ref[...]
    )


def _sc_body(xpad_hbm, asrc_hbm, adst_hbm, ei_hbm, part_hbm,
             ei_v, sa_v, sd_v, ex_v, rows_v, acc_sp, gsem, ssem, isem, zsem):
    c = lax.axis_index("c")
    s = lax.axis_index("s")
    wid = c * NSUB + s
    start = wid * NG

    z16f = jnp.zeros((16,), jnp.float32)
    z16i = jnp.zeros((16,), jnp.int32)

    for m in range(NI):
        for i in range(2):
            for j in range(CH // 16):
                ei_v[m, i, pl.ds(j * 16, 16)] = z16i

    def _zero_row(r, _):
        for b in range(NR):
            for j in range(DP // 16):
                rows_v[b, r, pl.ds(j * 16, 16)] = z16f
        return 0
    lax.fori_loop(0, CH, _zero_row, 0)

    base = s * RPT
    nz = RPT // CH
    for k in range(nz):
        pltpu.async_copy(rows_v.at[0], acc_sp.at[pl.ds(base + k * CH, CH)],
                         zsem)
    pltpu.async_copy(rows_v.at[1].at[pl.ds(0, RPT - nz * CH)],
                     acc_sp.at[pl.ds(base + nz * CH, RPT - nz * CH)], zsem)
    for k in range(nz):
        pltpu.make_async_copy(rows_v.at[0],
                              acc_sp.at[pl.ds(base + k * CH, CH)], zsem).wait()
    pltpu.make_async_copy(rows_v.at[1].at[pl.ds(0, RPT - nz * CH)],
                          acc_sp.at[pl.ds(base + nz * CH, RPT - nz * CH)],
                          zsem).wait()
    plsc.subcore_barrier()

    for m in range(5):
        pltpu.async_copy(ei_hbm.at[pl.ds(2 * (start + m), 2)], ei_v.at[m], isem)
    for m in range(5):
        pltpu.make_async_copy(ei_hbm.at[pl.ds(2 * (start + m), 2)],
                              ei_v.at[m], isem).wait()
    for m in range(3):
        em = ei_v.at[m]
        pltpu.async_copy(xpad_hbm.at[em.at[0]], rows_v.at[m], gsem)
        pltpu.async_copy(asrc_hbm.at[em.at[0]], sa_v.at[m], gsem)
        pltpu.async_copy(adst_hbm.at[em.at[1]], sd_v.at[m], gsem)
        pltpu.async_copy(rows_v.at[3 + m], acc_sp.at[ei_v.at[11].at[1]],
                         ssem, add=True)

    def _block(p, _):
        g0 = start + p * 12
        for k in range(12):
            b = k % NR
            g = g0 + k
            gi = p * 12 + k
            ek = ei_v.at[k]
            pltpu.make_async_copy(
                xpad_hbm.at[ek.at[0]], rows_v.at[b], gsem).wait()
            pltpu.make_async_copy(asrc_hbm.at[ek.at[0]], sa_v.at[b], gsem).wait()
            pltpu.make_async_copy(adst_hbm.at[ek.at[1]], sd_v.at[b], gsem).wait()
            for j in range(CH // 16):
                ee = sa_v[b, pl.ds(j * 16, 16)] + sd_v[b, pl.ds(j * 16, 16)]
                ee = jnp.where(ee >= 0.0, ee, ee * 0.01)
                ex_v[pl.ds(j * 16, 16)] = jnp.exp(ee)
            pltpu.make_async_copy(
                rows_v.at[(b + 3) % NR],
                acc_sp.at[ei_v.at[(k + 9) % NI].at[1]], ssem).wait()

            @pl.when(gi + 3 < NG)
            def _():
                kn = (k + 3) % NI
                en = ei_v.at[kn]

                @pl.when(gi >= 2)
                def _():
                    pltpu.make_async_copy(
                        ei_hbm.at[pl.ds(2 * (g + 3), 2)], en, isem).wait()
                pltpu.async_copy(
                    xpad_hbm.at[en.at[0]], rows_v.at[(b + 3) % NR], gsem)
                pltpu.async_copy(asrc_hbm.at[en.at[0]], sa_v.at[(b + 3) % NR],
                                 gsem)
                pltpu.async_copy(adst_hbm.at[en.at[1]], sd_v.at[(b + 3) % NR],
                                 gsem)

            @pl.when(gi + 5 < NG)
            def _():
                pltpu.async_copy(ei_hbm.at[pl.ds(2 * (g + 5), 2)],
                                 ei_v.at[(k + 5) % NI], isem)

            def _mul_row(r, _2):
                ev = plsc.load_gather(ex_v, [jnp.full((16,), r, jnp.int32)])
                for j in range(DP // 16):
                    rows_v[b, r, pl.ds(j * 16, 16)] = (
                        rows_v[b, r, pl.ds(j * 16, 16)] * ev)
                return 0
            lax.fori_loop(0, CH, _mul_row, 0)

            pltpu.async_copy(rows_v.at[b], acc_sp.at[ek.at[1]], ssem, add=True)
        return 0
    lax.fori_loop(0, NQ, _block, 0)

    for m in range(3):
        pltpu.make_async_copy(
            rows_v.at[(NG - 3 + m) % NR],
            acc_sp.at[ei_v.at[(NG - 3 + m) % NI].at[1]], ssem).wait()
    plsc.subcore_barrier()
    pltpu.sync_copy(acc_sp.at[pl.ds(base, RPT)],
                    part_hbm.at[c, pl.ds(base, RPT)])


def kernel(x, edge_index, Wc, Wq, Wk, W_attn, Wfc, b_fc):
    del Wq, Wk
    src = edge_index[0].astype(jnp.int32)
    dst = edge_index[1].astype(jnp.int32)
    pad = jnp.full((EPAD - E,), N, jnp.int32)
    ei = jnp.concatenate([jnp.concatenate([src, pad]).reshape(NCHP, CH),
                          jnp.concatenate([dst, pad]).reshape(NCHP, CH)],
                         axis=1).reshape(2 * NCHP, CH)
    xin = jnp.concatenate([x, jnp.zeros((NPAD - N, D), jnp.float32)], axis=0)
    c12 = jnp.stack([Wc @ W_attn[:COUT, 0], Wc @ W_attn[COUT:, 0]], axis=1)

    xpad, atab = pl.pallas_call(
        _prep_body,
        grid=(NPAD // BNP,),
        in_specs=[
            pl.BlockSpec((BNP, D), lambda i: (i, 0)),
            pl.BlockSpec((D, 2), lambda i: (0, 0)),
        ],
        out_specs=[
            pl.BlockSpec((BNP, DP), lambda i: (i, 0)),
            pl.BlockSpec((BNP, 2), lambda i: (i, 0)),
        ],
        out_shape=[
            jax.ShapeDtypeStruct((NPAD, DP), jnp.float32),
            jax.ShapeDtypeStruct((NPAD, 2), jnp.float32),
        ],
    )(xin, c12)
    asrc = atab[:, 0]
    adst = atab[:, 1]

    mesh = plsc.VectorSubcoreMesh(core_axis_name="c", subcore_axis_name="s")
    part = pl.kernel(
        _sc_body,
        out_type=jax.ShapeDtypeStruct((NCORES, NPAD, DP), jnp.float32),
        mesh=mesh,
        compiler_params=pltpu.CompilerParams(needs_layout_passes=False,
                                             use_tc_tiling_on_sc=False),
        scratch_types=[
            pltpu.VMEM((NI, 2, CH), jnp.int32),
            pltpu.VMEM((NR, CH), jnp.float32),
            pltpu.VMEM((NR, CH), jnp.float32),
            pltpu.VMEM((CH,), jnp.float32),
            pltpu.VMEM((NR, CH, DP), jnp.float32),
            pltpu.VMEM_SHARED((NPAD, DP), jnp.float32),
            pltpu.SemaphoreType.DMA,
            pltpu.SemaphoreType.DMA,
            pltpu.SemaphoreType.DMA,
            pltpu.SemaphoreType.DMA,
        ],
    )(xpad, asrc, adst, ei)

    out = pl.pallas_call(
        _finish_body,
        grid=(N // BN,),
        in_specs=[
            pl.BlockSpec((NCORES, BN, DP), lambda i: (0, i, 0)),
            pl.BlockSpec((D, D), lambda i: (0, 0)),
            pl.BlockSpec((1, D), lambda i: (0, 0)),
        ],
        out_specs=pl.BlockSpec((BN, D), lambda i: (i, 0)),
        out_shape=jax.ShapeDtypeStruct((N, D), jnp.float32),
    )(part, Wfc, b_fc.reshape(1, D))
    return out


# speedup vs baseline: 1.0415x; 1.0415x over previous
import jax
import jax.numpy as jnp
from jax import lax
from jax.experimental import pallas as pl
from jax.experimental.pallas import tpu as pltpu
from jax.experimental.pallas import tpu_sc as plsc

N = 10000
E = 320000
D = 128
COUT = 64
DP = 144
CH = 128
NCORES = 2
NSUB = 16
NTILES = NCORES * NSUB
NG = 80
EPAD = NTILES * NG * CH
NCHP = EPAD // CH
NPAD = 10112
RPT = NPAD // NSUB
BNP = 1264
BN = 1000


def _prep_body(x_ref, c12_ref, xpad_ref, atab_ref):
    xb = x_ref[...]
    ones = jnp.ones((BNP, 1), jnp.float32)
    zeros = jnp.zeros((BNP, DP - D - 1), jnp.float32)
    xpad_ref[...] = jnp.concatenate([xb, ones, zeros], axis=1)
    atab_ref[...] = jnp.dot(xb, c12_ref[...], preferred_element_type=jnp.float32)


def _finish_body(part_ref, wfc_ref, b_ref, out_ref):
    s = part_ref[0] + part_ref[1]
    h = s[:, :D]
    den = s[:, D:D + 1]
    h = h * (1.0 / (den + 1e-9))
    out_ref[...] = (
        jnp.dot(h, wfc_ref[...], preferred_element_type=jnp.float32) + b_ref[...]
    )


def _sc_body(xpad_hbm, atab_hbm, ei_hbm, part_hbm,
             ei_v, atab_v, ex_v, rows_v, acc_sp, gsem, zsem):
    c = lax.axis_index("c")
    s = lax.axis_index("s")
    wid = c * NSUB + s
    start = wid * NG

    z16f = jnp.zeros((16,), jnp.float32)
    one16 = jnp.ones((16,), jnp.int32)

    pltpu.sync_copy(atab_hbm, atab_v)

    def _zero_row(r, _):
        for j in range(DP // 16):
            rows_v[r, pl.ds(j * 16, 16)] = z16f
        return 0
    lax.fori_loop(0, CH, _zero_row, 0)

    base = s * RPT
    nz = RPT // CH
    for k in range(nz):
        pltpu.async_copy(rows_v, acc_sp.at[pl.ds(base + k * CH, CH)], zsem)
    pltpu.async_copy(rows_v.at[pl.ds(0, RPT - nz * CH)],
                     acc_sp.at[pl.ds(base + nz * CH, RPT - nz * CH)], zsem)
    for k in range(nz):
        pltpu.make_async_copy(rows_v, acc_sp.at[pl.ds(base + k * CH, CH)],
                              zsem).wait()
    pltpu.make_async_copy(rows_v.at[pl.ds(0, RPT - nz * CH)],
                          acc_sp.at[pl.ds(base + nz * CH, RPT - nz * CH)],
                          zsem).wait()
    plsc.subcore_barrier()

    def _chunk(i, _):
        g = start + i
        pltpu.sync_copy(ei_hbm.at[pl.ds(2 * g, 2)], ei_v)
        for j in range(CH // 16):
            sv = ei_v[0, pl.ds(j * 16, 16)]
            dv = ei_v[1, pl.ds(j * 16, 16)]
            a1 = plsc.load_gather(atab_v, [sv * 2])
            a2 = plsc.load_gather(atab_v, [dv * 2 + one16])
            ee = a1 + a2
            ee = jnp.where(ee >= 0.0, ee, ee * 0.01)
            ex_v[pl.ds(j * 16, 16)] = jnp.exp(ee)
        pltpu.async_copy(xpad_hbm.at[ei_v.at[0]], rows_v, gsem).wait()

        def _mul_row(r, _2):
            ev = plsc.load_gather(ex_v, [jnp.full((16,), r, jnp.int32)])
            for j in range(DP // 16):
                rows_v[r, pl.ds(j * 16, 16)] = rows_v[r, pl.ds(j * 16, 16)] * ev
            return 0
        lax.fori_loop(0, CH, _mul_row, 0)

        pltpu.sync_copy(rows_v, acc_sp.at[ei_v.at[1]], add=True)
        return 0
    lax.fori_loop(0, NG, _chunk, 0)

    plsc.subcore_barrier()
    pltpu.sync_copy(acc_sp.at[pl.ds(base, RPT)],
                    part_hbm.at[c, pl.ds(base, RPT)])


def kernel(x, edge_index, Wc, Wq, Wk, W_attn, Wfc, b_fc):
    del Wq, Wk
    src = edge_index[0].astype(jnp.int32)
    dst = edge_index[1].astype(jnp.int32)
    pad = jnp.full((EPAD - E,), N, jnp.int32)
    ei = jnp.concatenate([jnp.concatenate([src, pad]).reshape(NCHP, CH),
                          jnp.concatenate([dst, pad]).reshape(NCHP, CH)],
                         axis=1).reshape(2 * NCHP, CH)
    xin = jnp.concatenate([x, jnp.zeros((NPAD - N, D), jnp.float32)], axis=0)
    c12 = jnp.stack([Wc @ W_attn[:COUT, 0], Wc @ W_attn[COUT:, 0]], axis=1)

    xpad, atab = pl.pallas_call(
        _prep_body,
        grid=(NPAD // BNP,),
        in_specs=[
            pl.BlockSpec((BNP, D), lambda i: (i, 0)),
            pl.BlockSpec((D, 2), lambda i: (0, 0)),
        ],
        out_specs=[
            pl.BlockSpec((BNP, DP), lambda i: (i, 0)),
            pl.BlockSpec((BNP, 2), lambda i: (i, 0)),
        ],
        out_shape=[
            jax.ShapeDtypeStruct((NPAD, DP), jnp.float32),
            jax.ShapeDtypeStruct((NPAD, 2), jnp.float32),
        ],
    )(xin, c12)

    mesh = plsc.VectorSubcoreMesh(core_axis_name="c", subcore_axis_name="s")
    part = pl.kernel(
        _sc_body,
        out_type=jax.ShapeDtypeStruct((NCORES, NPAD, DP), jnp.float32),
        mesh=mesh,
        compiler_params=pltpu.CompilerParams(needs_layout_passes=False,
                                             use_tc_tiling_on_sc=False),
        scratch_types=[
            pltpu.VMEM((2, CH), jnp.int32),
            pltpu.VMEM((2 * NPAD,), jnp.float32),
            pltpu.VMEM((CH,), jnp.float32),
            pltpu.VMEM((CH, DP), jnp.float32),
            pltpu.VMEM_SHARED((NPAD, DP), jnp.float32),
            pltpu.SemaphoreType.DMA,
            pltpu.SemaphoreType.DMA,
        ],
    )(xpad, atab.reshape(2 * NPAD), ei)

    out = pl.pallas_call(
        _finish_body,
        grid=(N // BN,),
        in_specs=[
            pl.BlockSpec((NCORES, BN, DP), lambda i: (0, i, 0)),
            pl.BlockSpec((D, D), lambda i: (0, 0)),
            pl.BlockSpec((1, D), lambda i: (0, 0)),
        ],
        out_specs=pl.BlockSpec((BN, D), lambda i: (i, 0)),
        out_shape=jax.ShapeDtypeStruct((N, D), jnp.float32),
    )(part, Wfc, b_fc.reshape(1, D))
    return out
